# baseline (speedup 1.0000x reference)
import functools

import jax
import jax.numpy as jnp
from jax.experimental import pallas as pl
from jax.experimental.pallas import tpu as pltpu


def _gcn_body(y_ref, w1_ref, b1_ref, w2_ref, b2_ref, m1_ref, c1_ref,
              m2_ref, c2_ref, out_ref, r_scr, abf_scr, dinv_scr, lw_scr,
              *, n_batch):
    b = pl.program_id(0)
    n = y_ref.shape[1]
    nh = w1_ref.shape[0]
    b1c = b1_ref[...].reshape(nh, 1)
    b2c = b2_ref[...].reshape(nh, 1)
    c1r = c1_ref[...].reshape(1, -1)
    c2r = c2_ref[...].reshape(1, -1)
    vb = jnp.minimum(b, n_batch - 1)
    mb = jnp.maximum(b - 1, 0)

    a = y_ref[0]
    tile = 128
    row_i = jax.lax.broadcasted_iota(jnp.int32, (tile, tile), 0)
    col_i = jax.lax.broadcasted_iota(jnp.int32, (tile, tile), 1)
    mask = row_i == col_i
    diag = jnp.concatenate(
        [jnp.sum(jnp.where(mask,
                           y_ref[0, t * tile:(t + 1) * tile,
                                 t * tile:(t + 1) * tile], 0.0),
                 axis=0, keepdims=True)
         for t in range(n // tile)], axis=1)
    loop_w = jnp.where(diag == 0.0, 1.0, 0.0)
    deg = jnp.sum(a, axis=0, keepdims=True) + loop_w
    dinv = jnp.where(deg > 0.0, jax.lax.rsqrt(jnp.where(deg > 0.0, deg, 1.0)),
                     0.0)
    abf_scr[pl.ds(vb, 1)] = a.astype(jnp.bfloat16).reshape(1, n, n)
    dinv_scr[pl.ds(vb, 1), :] = dinv
    lw_scr[pl.ds(vb, 1), :] = loop_w

    dinv_m = dinv_scr[pl.ds(mb, 1), :]
    lw_m = lw_scr[pl.ds(mb, 1), :]
    abf = abf_scr[pl.ds(mb, 1)].reshape(n, n)

    t1 = jnp.dot(dinv_m.astype(jnp.bfloat16), abf,
                 preferred_element_type=jnp.float32)
    s = dinv_m * (t1 + dinv_m * lw_m)
    x1t = jnp.maximum(w1_ref[...] * s + b1c, 0.0)

    h2t = jnp.dot(w2_ref[...], x1t,
                  preferred_element_type=jnp.float32)
    gt = dinv_m * h2t
    zt = jnp.dot(gt.astype(jnp.bfloat16), abf,
                 preferred_element_type=jnp.float32)
    out2t = dinv_m * (zt + lw_m * gt) + b2c
    r_scr[pl.ds(mb, 1), :] = jnp.max(out2t, axis=0, keepdims=True)

    @pl.when(b == n_batch)
    def _():
        rr = r_scr[...]
        o1 = jax.lax.dot_general(
            rr, m1_ref[...], (((1,), (1,)), ((), ())),
            preferred_element_type=jnp.float32) + c1r
        o2 = jax.lax.dot_general(
            o1, m2_ref[...], (((1,), (1,)), ((), ())),
            preferred_element_type=jnp.float32) + c2r
        out_ref[...] = o2


def kernel(y, W1, b1, W2, b2, M1, c1, M2, c2):
    B, N = y.shape[0], y.shape[1]

    vmem = pl.BlockSpec(memory_space=pltpu.MemorySpace.VMEM)
    return pl.pallas_call(
        functools.partial(_gcn_body, n_batch=B),
        grid=(B + 1,),
        in_specs=[
            pl.BlockSpec((1, N, N),
                         lambda b, _last=B - 1: (jnp.minimum(b, _last), 0, 0)),
            vmem, vmem, vmem, vmem, vmem, vmem, vmem, vmem,
        ],
        out_specs=pl.BlockSpec((B, M2.shape[0]), lambda b: (0, 0)),
        out_shape=jax.ShapeDtypeStruct((B, M2.shape[0]), jnp.float32),
        scratch_shapes=[
            pltpu.VMEM((B, N), jnp.float32),
            pltpu.VMEM((B, N, N), jnp.bfloat16),
            pltpu.VMEM((B, N), jnp.float32),
            pltpu.VMEM((B, N), jnp.float32),
        ],
    )(y, W1, b1, W2, b2, M1, c1, M2, c2)


# speedup vs baseline: 1.1558x; 1.1558x over previous
import jax
import jax.numpy as jnp
from jax.experimental import pallas as pl
from jax.experimental.pallas import tpu as pltpu


def _gcn_body(y_ref, w1_ref, b1_ref, w2_ref, b2_ref, m1_ref, c1_ref,
              m2_ref, c2_ref, out_ref):
    n_batch, n = y_ref.shape[0], y_ref.shape[1]
    nh = w1_ref.shape[0]
    b1c = b1_ref[...].reshape(nh, 1)
    b2c = b2_ref[...].reshape(nh, 1)
    c1r = c1_ref[...].reshape(1, -1)
    c2r = c2_ref[...].reshape(1, -1)

    tile = 128
    row_i = jax.lax.broadcasted_iota(jnp.int32, (tile, tile), 0)
    col_i = jax.lax.broadcasted_iota(jnp.int32, (tile, tile), 1)
    mask = row_i == col_i

    r_rows = []
    for b in range(n_batch):
        a = y_ref[b]
        diag = jnp.concatenate(
            [jnp.sum(jnp.where(mask,
                               y_ref[b, t * tile:(t + 1) * tile,
                                     t * tile:(t + 1) * tile], 0.0),
                     axis=0, keepdims=True)
             for t in range(n // tile)], axis=1)
        loop_w = jnp.where(diag == 0.0, 1.0, 0.0)
        deg = jnp.sum(a, axis=0, keepdims=True) + loop_w
        dinv = jnp.where(deg > 0.0,
                         jax.lax.rsqrt(jnp.where(deg > 0.0, deg, 1.0)), 0.0)

        a_bf = a.astype(jnp.bfloat16)

        t1 = jnp.dot(dinv.astype(jnp.bfloat16), a_bf,
                     preferred_element_type=jnp.float32)
        s = dinv * (t1 + dinv * loop_w)
        x1t = jnp.maximum(w1_ref[...] * s + b1c, 0.0)

        h2t = jnp.dot(w2_ref[...], x1t,
                      preferred_element_type=jnp.float32)
        gt = dinv * h2t
        zt = jnp.dot(gt.astype(jnp.bfloat16), a_bf,
                     preferred_element_type=jnp.float32)
        out2t = dinv * (zt + loop_w * gt) + b2c
        r_rows.append(jnp.max(out2t, axis=0, keepdims=True))

    rr = jnp.concatenate(r_rows, axis=0)
    o1 = jax.lax.dot_general(
        rr, m1_ref[...], (((1,), (1,)), ((), ())),
        preferred_element_type=jnp.float32) + c1r
    o2 = jax.lax.dot_general(
        o1, m2_ref[...], (((1,), (1,)), ((), ())),
        preferred_element_type=jnp.float32) + c2r
    out_ref[...] = o2


def kernel(y, W1, b1, W2, b2, M1, c1, M2, c2):
    B = y.shape[0]
    vmem = pl.BlockSpec(memory_space=pltpu.MemorySpace.VMEM)
    return pl.pallas_call(
        _gcn_body,
        in_specs=[vmem] * 9,
        out_specs=vmem,
        out_shape=jax.ShapeDtypeStruct((B, M2.shape[0]), jnp.float32),
    )(y, W1, b1, W2, b2, M1, c1, M2, c2)


# speedup vs baseline: 1.2008x; 1.0389x over previous
import functools

import jax
import jax.numpy as jnp
from jax.experimental import pallas as pl
from jax.experimental.pallas import tpu as pltpu


def _gcn_body(y_ref, w1_ref, b1_ref, w2_ref, b2_ref, m1_ref, c1_ref,
              m2_ref, c2_ref, out_ref, r_scr, m12_scr, bh_scr, *, n_batch):
    b = pl.program_id(0)
    a = y_ref[0]
    n = a.shape[0]
    nh = w1_ref.shape[0]
    b1c = b1_ref[...].reshape(nh, 1)
    b2c = b2_ref[...].reshape(nh, 1)
    c1r = c1_ref[...].reshape(1, -1)
    c2r = c2_ref[...].reshape(1, -1)

    @pl.when(b == 0)
    def _():
        m12_scr[...] = jnp.dot(m2_ref[...], m1_ref[...],
                               preferred_element_type=jnp.float32)
        bh_scr[...] = jax.lax.dot_general(
            c1r, m2_ref[...], (((1,), (1,)), ((), ())),
            preferred_element_type=jnp.float32) + c2r

    tile = 128
    row_i = jax.lax.broadcasted_iota(jnp.int32, (tile, tile), 0)
    col_i = jax.lax.broadcasted_iota(jnp.int32, (tile, tile), 1)
    mask = row_i == col_i
    diag = jnp.concatenate(
        [jnp.sum(jnp.where(mask,
                           y_ref[0, t * tile:(t + 1) * tile,
                                 t * tile:(t + 1) * tile], 0.0),
                 axis=0, keepdims=True)
         for t in range(n // tile)], axis=1)
    loop_w = jnp.where(diag == 0.0, 1.0, 0.0)
    deg = jnp.sum(a, axis=0, keepdims=True) + loop_w
    dinv = jnp.where(deg > 0.0, jax.lax.rsqrt(jnp.where(deg > 0.0, deg, 1.0)),
                     0.0)

    a_bf = a.astype(jnp.bfloat16)

    t1 = jnp.dot(dinv.astype(jnp.bfloat16), a_bf,
                 preferred_element_type=jnp.float32)
    s = dinv * (t1 + dinv * loop_w)
    x1t = jnp.maximum(w1_ref[...] * s + b1c, 0.0)

    h2t = jnp.dot(w2_ref[...], x1t,
                  preferred_element_type=jnp.float32)
    gt = dinv * h2t
    zt = jnp.dot(gt.astype(jnp.bfloat16), a_bf,
                 preferred_element_type=jnp.float32)
    out2t = dinv * (zt + loop_w * gt) + b2c
    r_scr[pl.ds(b, 1), :] = jnp.max(out2t, axis=0, keepdims=True)

    @pl.when(b == n_batch - 1)
    def _():
        rr = r_scr[...]
        out_ref[...] = jax.lax.dot_general(
            rr, m12_scr[...], (((1,), (1,)), ((), ())),
            preferred_element_type=jnp.float32) + bh_scr[...]


def kernel(y, W1, b1, W2, b2, M1, c1, M2, c2):
    B, N = y.shape[0], y.shape[1]

    vmem = pl.BlockSpec(memory_space=pltpu.MemorySpace.VMEM)
    return pl.pallas_call(
        functools.partial(_gcn_body, n_batch=B),
        grid=(B,),
        in_specs=[
            pl.BlockSpec((1, N, N), lambda b: (b, 0, 0)),
            vmem, vmem, vmem, vmem, vmem, vmem, vmem, vmem,
        ],
        out_specs=pl.BlockSpec((B, M2.shape[0]), lambda b: (0, 0)),
        out_shape=jax.ShapeDtypeStruct((B, M2.shape[0]), jnp.float32),
        scratch_shapes=[pltpu.VMEM((B, N), jnp.float32),
                        pltpu.VMEM((M2.shape[0], N), jnp.float32),
                        pltpu.VMEM((1, M2.shape[0]), jnp.float32)],
    )(y, W1, b1, W2, b2, M1, c1, M2, c2)


# speedup vs baseline: 1.2909x; 1.0751x over previous
import functools

import jax
import jax.numpy as jnp
from jax.experimental import pallas as pl
from jax.experimental.pallas import tpu as pltpu


def _gcn_body(y_ref, w1_ref, b1_ref, w2_ref, b2_ref, m1_ref, c1_ref,
              m2_ref, c2_ref, out_ref, r_scr, *, n_batch):
    b = pl.program_id(0)
    a = y_ref[0]
    n = a.shape[0]
    nh = w1_ref.shape[0]
    b1c = b1_ref[...].reshape(nh, 1)
    b2c = b2_ref[...].reshape(nh, 1)
    c1r = c1_ref[...].reshape(1, -1)
    c2r = c2_ref[...].reshape(1, -1)

    tile = 128
    row_i = jax.lax.broadcasted_iota(jnp.int32, (tile, tile), 0)
    col_i = jax.lax.broadcasted_iota(jnp.int32, (tile, tile), 1)
    mask = row_i == col_i
    diag = jnp.concatenate(
        [jnp.sum(jnp.where(mask,
                           y_ref[0, t * tile:(t + 1) * tile,
                                 t * tile:(t + 1) * tile], 0.0),
                 axis=0, keepdims=True)
         for t in range(n // tile)], axis=1)
    loop_w = jnp.where(diag == 0.0, 1.0, 0.0)
    deg = jnp.sum(a, axis=0, keepdims=True) + loop_w
    dinv = jnp.where(deg > 0.0, jax.lax.rsqrt(jnp.where(deg > 0.0, deg, 1.0)),
                     0.0)

    a_bf = a.astype(jnp.bfloat16)

    t1 = jnp.dot(dinv.astype(jnp.bfloat16), a_bf,
                 preferred_element_type=jnp.float32)
    s = dinv * (t1 + dinv * loop_w)
    x1t = jnp.maximum(w1_ref[...] * s + b1c, 0.0)

    h2t = jnp.dot(w2_ref[...], x1t,
                  preferred_element_type=jnp.float32)
    gt = dinv * h2t
    zt = jnp.dot(gt.astype(jnp.bfloat16), a_bf,
                 preferred_element_type=jnp.float32)
    out2t = dinv * (zt + loop_w * gt) + b2c
    r_scr[pl.ds(b, 1), :] = jnp.max(out2t, axis=0, keepdims=True)

    @pl.when(b == n_batch - 1)
    def _():
        rr = r_scr[...]
        o1 = jax.lax.dot_general(
            rr, m1_ref[...], (((1,), (1,)), ((), ())),
            preferred_element_type=jnp.float32) + c1r
        o2 = jax.lax.dot_general(
            o1, m2_ref[...], (((1,), (1,)), ((), ())),
            preferred_element_type=jnp.float32) + c2r
        out_ref[...] = o2


def kernel(y, W1, b1, W2, b2, M1, c1, M2, c2):
    B, N = y.shape[0], y.shape[1]

    vmem = pl.BlockSpec(memory_space=pltpu.MemorySpace.VMEM)
    return pl.pallas_call(
        functools.partial(_gcn_body, n_batch=B),
        grid=(B,),
        in_specs=[
            pl.BlockSpec((1, N, N), lambda b: (b, 0, 0)),
            vmem, vmem, vmem, vmem, vmem, vmem, vmem, vmem,
        ],
        out_specs=pl.BlockSpec((B, M2.shape[0]), lambda b: (0, 0)),
        out_shape=jax.ShapeDtypeStruct((B, M2.shape[0]), jnp.float32),
        scratch_shapes=[pltpu.VMEM((B, N), jnp.float32)],
    )(y, W1, b1, W2, b2, M1, c1, M2, c2)
